# baseline (speedup 1.0000x reference)
import functools

import jax
import jax.numpy as jnp
from jax import lax
from jax.experimental import pallas as pl
from jax.experimental.pallas import tpu as pltpu
from jax.experimental.pallas import tpu_sc as plsc

NUM_WORKERS = 32
R_ITEMS = 2
NBUF = 2
PPI = 100
PAD = 104
LANES = 16


def _make_gather(batch, hist, d_model):
    per_w = batch // NUM_WORKERS
    n_ch = per_w // R_ITEMS
    assert per_w % (R_ITEMS * NBUF) == 0 and hist == 2 * PPI
    mesh = plsc.VectorSubcoreMesh(core_axis_name="c", subcore_axis_name="s")

    @functools.partial(
        pl.kernel,
        out_type=jax.ShapeDtypeStruct((batch, hist, d_model), jnp.float32),
        mesh=mesh,
        scratch_types=[
            pltpu.VMEM((NBUF, R_ITEMS, PAD), jnp.int32),
            pltpu.VMEM((NBUF, R_ITEMS, PAD, 2 * d_model), jnp.float32),
            pltpu.VMEM((NBUF, R_ITEMS, hist, d_model), jnp.float32),
            pltpu.SemaphoreType.DMA,
            pltpu.SemaphoreType.DMA,
            pltpu.SemaphoreType.DMA,
            pltpu.SemaphoreType.DMA,
        ],
        compiler_params=pltpu.CompilerParams(use_tc_tiling_on_sc=False),
    )
    def k(table_hbm, idx_hbm, out_hbm, idx_v, g_v, s_v, gsem0, gsem1, osem0, osem1):
        gsems = (gsem0, gsem1)
        osems = (osem0, osem1)
        wid = lax.axis_index("s") * 2 + lax.axis_index("c")
        base = wid * per_w

        def fire(g, b):
            i0 = base + g * R_ITEMS
            pltpu.sync_copy(idx_hbm.at[pl.ds(i0, R_ITEMS)], idx_v.at[b])
            for r in range(R_ITEMS):
                pltpu.async_copy(
                    table_hbm.at[idx_v.at[b].at[r]], g_v.at[b].at[r], gsems[b]
                )

        def wait_gathers(b):
            for r in range(R_ITEMS):
                pltpu.make_async_copy(
                    table_hbm.at[idx_v.at[b].at[r]], g_v.at[b].at[r], gsems[b]
                ).wait()

        def drain_out(b):
            pltpu.make_async_copy(
                s_v.at[b], out_hbm.at[pl.ds(0, R_ITEMS)], osems[b]
            ).wait()

        def vcopy(b):
            for r in range(R_ITEMS):
                def body_q(q, carry):
                    for c in range(2 * d_model // LANES):
                        h, kk = divmod(c * LANES, d_model)
                        s_v.at[b].at[r][q + PPI * h, pl.ds(kk, LANES)] = (
                            g_v.at[b].at[r][q, pl.ds(c * LANES, LANES)]
                        )
                    return carry
                lax.fori_loop(0, PPI, body_q, 0)

        fire(0, 0)

        def outer(t2, carry):
            for b in range(NBUF):
                g = t2 * NBUF + b

                @pl.when(g + 1 < n_ch)
                def _fire_next():
                    fire(g + 1, b ^ 1)

                wait_gathers(b)

                @pl.when(t2 > 0)
                def _wait_prev_out():
                    drain_out(b)

                vcopy(b)
                i0 = base + g * R_ITEMS
                pltpu.async_copy(
                    s_v.at[b], out_hbm.at[pl.ds(i0, R_ITEMS)], osems[b]
                )
            return carry

        lax.fori_loop(0, n_ch // NBUF, outer, 0)
        for b in range(NBUF):
            drain_out(b)

    return k


def kernel(cumulative_positions, position_embeddings):
    b, h = cumulative_positions.shape
    v = position_embeddings.shape[0]
    d = position_embeddings.shape[1]
    idx = cumulative_positions.astype(jnp.int32)
    pair_idx = idx[:, : h // 2] * v + idx[:, h // 2 :]
    pair_idx = jnp.pad(pair_idx, ((0, 0), (0, PAD - PPI)))
    left = jnp.broadcast_to(position_embeddings[:, None, :], (v, v, d))
    right = jnp.broadcast_to(position_embeddings[None, :, :], (v, v, d))
    t2 = jnp.concatenate([left, right], axis=-1).reshape(v * v, 2 * d)
    return _make_gather(b, h, d)(t2, pair_idx)


# speedup vs baseline: 1.0129x; 1.0129x over previous
import functools

import jax
import jax.numpy as jnp
from jax import lax
from jax.experimental import pallas as pl
from jax.experimental.pallas import tpu as pltpu
from jax.experimental.pallas import tpu_sc as plsc

NUM_WORKERS = 32
R_ITEMS = 2
NBUF = 2
PPI = 100
PAD = 104
LANES = 16


def _make_gather(batch, hist, d_model):
    per_w = batch // NUM_WORKERS
    n_ch = per_w // R_ITEMS
    assert per_w % (R_ITEMS * NBUF) == 0 and hist == 2 * PPI
    mesh = plsc.VectorSubcoreMesh(core_axis_name="c", subcore_axis_name="s")

    @functools.partial(
        pl.kernel,
        out_type=jax.ShapeDtypeStruct((batch, hist, d_model), jnp.float32),
        mesh=mesh,
        scratch_types=[
            pltpu.VMEM((NBUF, R_ITEMS, PAD), jnp.int32),
            pltpu.VMEM((NBUF, R_ITEMS, PAD, 2 * d_model), jnp.float32),
            pltpu.VMEM((NBUF, R_ITEMS, hist, d_model), jnp.float32),
            pltpu.SemaphoreType.DMA,
            pltpu.SemaphoreType.DMA,
            pltpu.SemaphoreType.DMA,
            pltpu.SemaphoreType.DMA,
        ],
        compiler_params=pltpu.CompilerParams(use_tc_tiling_on_sc=False),
    )
    def k(table_hbm, idx_hbm, out_hbm, idx_v, g_v, s_v, gsem0, gsem1, osem0, osem1):
        gsems = (gsem0, gsem1)
        osems = (osem0, osem1)
        wid = lax.axis_index("s") * 2 + lax.axis_index("c")
        base = wid * per_w

        def fire(g, b):
            i0 = base + g * R_ITEMS
            pltpu.sync_copy(idx_hbm.at[pl.ds(i0, R_ITEMS)], idx_v.at[b])
            for r in range(R_ITEMS):
                pltpu.async_copy(
                    table_hbm.at[idx_v.at[b].at[r]], g_v.at[b].at[r], gsems[b]
                )

        def wait_gathers(b):
            for r in range(R_ITEMS):
                pltpu.make_async_copy(
                    table_hbm.at[idx_v.at[b].at[r]], g_v.at[b].at[r], gsems[b]
                ).wait()

        def drain_out(b):
            pltpu.make_async_copy(
                s_v.at[b], out_hbm.at[pl.ds(0, R_ITEMS)], osems[b]
            ).wait()

        def vcopy(b):
            for r in range(R_ITEMS):
                @plsc.parallel_loop(0, PPI, unroll=4)
                def body_q(q):
                    for c in range(2 * d_model // LANES):
                        h, kk = divmod(c * LANES, d_model)
                        s_v.at[b].at[r][q + PPI * h, pl.ds(kk, LANES)] = (
                            g_v.at[b].at[r][q, pl.ds(c * LANES, LANES)]
                        )

        fire(0, 0)

        def outer(t2, carry):
            for b in range(NBUF):
                g = t2 * NBUF + b

                @pl.when(g + 1 < n_ch)
                def _fire_next():
                    fire(g + 1, b ^ 1)

                wait_gathers(b)

                @pl.when(t2 > 0)
                def _wait_prev_out():
                    drain_out(b)

                vcopy(b)
                i0 = base + g * R_ITEMS
                pltpu.async_copy(
                    s_v.at[b], out_hbm.at[pl.ds(i0, R_ITEMS)], osems[b]
                )
            return carry

        lax.fori_loop(0, n_ch // NBUF, outer, 0)
        for b in range(NBUF):
            drain_out(b)

    return k


def kernel(cumulative_positions, position_embeddings):
    b, h = cumulative_positions.shape
    v = position_embeddings.shape[0]
    d = position_embeddings.shape[1]
    idx = cumulative_positions.astype(jnp.int32)
    pair_idx = idx[:, : h // 2] * v + idx[:, h // 2 :]
    pair_idx = jnp.pad(pair_idx, ((0, 0), (0, PAD - PPI)))
    left = jnp.broadcast_to(position_embeddings[:, None, :], (v, v, d))
    right = jnp.broadcast_to(position_embeddings[None, :, :], (v, v, d))
    t2 = jnp.concatenate([left, right], axis=-1).reshape(v * v, 2 * d)
    return _make_gather(b, h, d)(t2, pair_idx)


# speedup vs baseline: 1.8390x; 1.8155x over previous
import functools

import jax
import jax.numpy as jnp
from jax import lax
from jax.experimental import pallas as pl
from jax.experimental.pallas import tpu as pltpu
from jax.experimental.pallas import tpu_sc as plsc

WIDTH = 128
NUM_WORKERS = 32
SUB = 128
K = 2
CHUNK = SUB * K
NBUF = 2


def _make_gather(m_rows):
    per_w = m_rows // NUM_WORKERS
    n_ch = per_w // CHUNK
    assert per_w % (CHUNK * NBUF) == 0
    mesh = plsc.VectorSubcoreMesh(core_axis_name="c", subcore_axis_name="s")

    @functools.partial(
        pl.kernel,
        out_type=jax.ShapeDtypeStruct((m_rows, WIDTH), jnp.float32),
        mesh=mesh,
        scratch_types=[
            pltpu.VMEM((NBUF, K, SUB), jnp.int32),
            pltpu.VMEM((NBUF, CHUNK, WIDTH), jnp.float32),
            pltpu.SemaphoreType.DMA,
            pltpu.SemaphoreType.DMA,
            pltpu.SemaphoreType.DMA,
        ],
        compiler_params=pltpu.CompilerParams(use_tc_tiling_on_sc=True),
    )
    def k(table_hbm, idx_hbm, out_hbm, idx_v, rows_v, gsem, osem0, osem1):
        osems = (osem0, osem1)
        wid = lax.axis_index("s") * 2 + lax.axis_index("c")
        base = wid * (per_w // SUB)

        def outer(t, carry):
            for b in range(NBUF):
                row0 = base + (t * NBUF + b) * K

                @pl.when(t > 0)
                def _wait_prev_scatter():
                    pltpu.make_async_copy(
                        rows_v.at[b], out_hbm.at[pl.ds(0, CHUNK)], osems[b]
                    ).wait()

                pltpu.sync_copy(idx_hbm.at[pl.ds(row0, K)], idx_v.at[b])
                descs = [
                    pltpu.async_copy(
                        table_hbm.at[idx_v.at[b].at[j]],
                        rows_v.at[b].at[pl.ds(j * SUB, SUB)],
                        gsem,
                    )
                    for j in range(K)
                ]
                for d in descs:
                    d.wait()
                pltpu.async_copy(
                    rows_v.at[b], out_hbm.at[pl.ds(row0 * SUB, CHUNK)], osems[b]
                )
            return carry

        lax.fori_loop(0, n_ch // NBUF, outer, 0)
        for b in range(NBUF):
            pltpu.make_async_copy(
                rows_v.at[b], out_hbm.at[pl.ds(0, CHUNK)], osems[b]
            ).wait()

    return k


def kernel(cumulative_positions, position_embeddings):
    b, h = cumulative_positions.shape
    n = b * h
    v = position_embeddings.shape[0]
    d = position_embeddings.shape[1]
    flat = cumulative_positions.astype(jnp.int32).reshape(n)
    pair_idx = flat[0::2] * v + flat[1::2]
    left = jnp.broadcast_to(position_embeddings[:, None, :], (v, v, d))
    right = jnp.broadcast_to(position_embeddings[None, :, :], (v, v, d))
    t2 = jnp.concatenate([left, right], axis=-1).reshape(v * v, 2 * d)
    idx2d = pair_idx.reshape(n // 2 // SUB, SUB)
    out = _make_gather(n // 2)(t2, idx2d)
    return out.reshape(b, h, d)
